# baseline (speedup 1.0000x reference)
import functools

import jax
import jax.numpy as jnp
from jax import lax
from jax.experimental import pallas as pl
from jax.experimental.pallas import tpu as pltpu
from jax.experimental.pallas import tpu_sc as plsc

_SC_PARAMS = pltpu.CompilerParams(use_tc_tiling_on_sc=False)

_NC = 2
_NS = 16
_BLK = 1024


_DEGW = 16


def _sc_degree(dstr, zeros1, ones1, n_pad):
    kd = dstr.shape[1]
    c = dstr.shape[2]
    stripe = n_pad // _NS
    mesh = plsc.VectorSubcoreMesh(core_axis_name="c", subcore_axis_name="s", num_cores=_NC, num_subcores=_NS)

    @functools.partial(
        pl.kernel,
        out_type=jax.ShapeDtypeStruct((_NC, n_pad, _DEGW), jnp.float32),
        mesh=mesh,
        compiler_params=_SC_PARAMS,
        scratch_types=[
            pltpu.VMEM((kd, c), jnp.int32),
            pltpu.VMEM((c, _DEGW), jnp.float32),
            pltpu.VMEM_SHARED((n_pad, _DEGW), jnp.float32),
        ],
    )
    def k(dstr_h, zeros_h, ones_h, out_h, didx, ones_v, acc):
        c = lax.axis_index("c")
        s = lax.axis_index("s")
        wid = s * _NC + c
        pltpu.sync_copy(zeros_h, acc.at[pl.ds(s * stripe, stripe)])
        pltpu.sync_copy(dstr_h.at[wid], didx)
        pltpu.sync_copy(ones_h, ones_v)
        plsc.subcore_barrier()

        def body(j, carry):
            pltpu.sync_copy(ones_v, acc.at[didx.at[j]], add=True)
            return carry

        lax.fori_loop(0, kd, body, 0)
        plsc.subcore_barrier()
        pltpu.sync_copy(acc.at[pl.ds(s * stripe, stripe)],
                        out_h.at[c, pl.ds(s * stripe, stripe)])

    return k(dstr, zeros1, ones1)


def _sc_scatter(y, srcr, dstr, zeros, n_pad, f, c):
    kd = srcr.shape[1]
    stripe = n_pad // _NS
    mesh = plsc.VectorSubcoreMesh(core_axis_name="c", subcore_axis_name="s", num_cores=_NC, num_subcores=_NS)

    @functools.partial(
        pl.kernel,
        out_type=jax.ShapeDtypeStruct((_NC, n_pad, f), jnp.float32),
        mesh=mesh,
        compiler_params=_SC_PARAMS,
        scratch_types=[
            pltpu.VMEM((kd, c), jnp.int32),
            pltpu.VMEM((kd, c), jnp.int32),
            pltpu.VMEM((2 * c, f), jnp.float32),
            pltpu.VMEM_SHARED((n_pad, f), jnp.float32),
            pltpu.SemaphoreType.DMA((2,)),
            pltpu.SemaphoreType.DMA((2,)),
        ],
    )
    def k(y_h, srcr_h, dstr_h, zeros_h, out_h, sidx, didx, rows, acc, gsem,
          ssem):
        cc = lax.axis_index("c")
        s = lax.axis_index("s")
        wid = s * _NC + cc
        pltpu.sync_copy(zeros_h, acc.at[pl.ds(s * stripe, stripe)])
        pltpu.sync_copy(srcr_h.at[wid], sidx)
        pltpu.sync_copy(dstr_h.at[wid], didx)
        plsc.subcore_barrier()

        def wait_scatter(sem_slot):
            pltpu.make_async_copy(rows.at[pl.ds(0, c)], acc.at[didx.at[0]],
                                  ssem.at[sem_slot]).wait()

        pltpu.async_copy(y_h.at[sidx.at[0]], rows.at[pl.ds(0, c)], gsem.at[0])

        def body(j, carry):
            par = lax.rem(j, 2)
            nxt = lax.rem(j + 1, 2)

            @pl.when(j >= 1)
            def _free():
                wait_scatter(nxt)

            @pl.when(j + 1 < kd)
            def _prefetch():
                pltpu.async_copy(y_h.at[sidx.at[j + 1]],
                                 rows.at[pl.ds(nxt * c, c)], gsem.at[nxt])

            pltpu.make_async_copy(y_h.at[sidx.at[0]], rows.at[pl.ds(0, c)],
                                  gsem.at[par]).wait()
            pltpu.async_copy(rows.at[pl.ds(par * c, c)], acc.at[didx.at[j]],
                             ssem.at[par], add=True)
            return carry

        lax.fori_loop(0, kd, body, 0)
        wait_scatter((kd - 1) % 2)
        plsc.subcore_barrier()
        pltpu.sync_copy(acc.at[pl.ds(s * stripe, stripe)],
                        out_h.at[cc, pl.ds(s * stripe, stripe)])

    return k(y, srcr, dstr, zeros)


def _tc_matmul(x, w):
    n, d = x.shape
    f = w.shape[1]

    def body(x_r, w_r, o_r):
        o_r[...] = jnp.dot(x_r[...], w_r[...], preferred_element_type=jnp.float32)

    return pl.pallas_call(
        body,
        grid=(n // _BLK,),
        in_specs=[pl.BlockSpec((_BLK, d), lambda i: (i, 0)),
                  pl.BlockSpec((d, f), lambda i: (0, 0))],
        out_specs=pl.BlockSpec((_BLK, f), lambda i: (i, 0)),
        out_shape=jax.ShapeDtypeStruct((n, f), jnp.float32),
    )(x, w)


def _tc_scale(degp, xw):
    n, d = xw.shape

    def body(dp_r, xw_r, y_r, di_r):
        dinv = lax.rsqrt(dp_r[0, :, 0:1] + dp_r[1, :, 0:1] + 1.0)
        di_r[...] = dinv
        y_r[...] = xw_r[...] * dinv

    return pl.pallas_call(
        body,
        grid=(n // _BLK,),
        in_specs=[pl.BlockSpec((2, _BLK, _DEGW), lambda i: (0, i, 0)),
                  pl.BlockSpec((_BLK, d), lambda i: (i, 0))],
        out_specs=[pl.BlockSpec((_BLK, d), lambda i: (i, 0)),
                   pl.BlockSpec((_BLK, 1), lambda i: (i, 0))],
        out_shape=[jax.ShapeDtypeStruct((n, d), jnp.float32),
                   jax.ShapeDtypeStruct((n, 1), jnp.float32)],
    )(degp, xw)


def _tc_mid(parts, y1, dinv, b1, w2):
    n, d = y1.shape
    f = w2.shape[1]

    def body(p_r, y_r, di_r, b_r, w_r, o_r):
        agg = p_r[0] + p_r[1] + y_r[...]
        h = jnp.maximum(agg * di_r[...] + b_r[...], 0.0)
        o_r[...] = jnp.dot(h, w_r[...], preferred_element_type=jnp.float32) * di_r[...]

    return pl.pallas_call(
        body,
        grid=(n // _BLK,),
        in_specs=[pl.BlockSpec((2, _BLK, d), lambda i: (0, i, 0)),
                  pl.BlockSpec((_BLK, d), lambda i: (i, 0)),
                  pl.BlockSpec((_BLK, 1), lambda i: (i, 0)),
                  pl.BlockSpec((1, d), lambda i: (0, 0)),
                  pl.BlockSpec((d, f), lambda i: (0, 0))],
        out_specs=pl.BlockSpec((_BLK, f), lambda i: (i, 0)),
        out_shape=jax.ShapeDtypeStruct((n, f), jnp.float32),
    )(parts, y1, dinv, b1.reshape(1, -1), w2)


def _tc_out(parts, y2, dinv, b2, n_out):
    n, f = y2.shape

    def body(p_r, y_r, di_r, b_r, o_r):
        o = (p_r[0] + p_r[1] + y_r[...]) * di_r[...] + b_r[...]
        m = jnp.max(o, axis=-1, keepdims=True)
        lse = jnp.log(jnp.sum(jnp.exp(o - m), axis=-1, keepdims=True)) + m
        o_r[...] = o - lse

    return pl.pallas_call(
        body,
        grid=(n // _BLK,),
        in_specs=[pl.BlockSpec((2, _BLK, f), lambda i: (0, i, 0)),
                  pl.BlockSpec((_BLK, f), lambda i: (i, 0)),
                  pl.BlockSpec((_BLK, 1), lambda i: (i, 0)),
                  pl.BlockSpec((1, f), lambda i: (0, 0))],
        out_specs=pl.BlockSpec((_BLK, f), lambda i: (i, 0)),
        out_shape=jax.ShapeDtypeStruct((n_out, f), jnp.float32),
    )(parts, y2, dinv, b2.reshape(1, -1))


def kernel(x, edge_index, W1, b1, W2, b2):
    n, d_in = x.shape
    e = edge_index.shape[1]
    hid = W1.shape[1]
    out_dim = W2.shape[1]

    n_pad = ((n + 2047) // 2048) * 2048
    nw = _NC * _NS
    stripe = n_pad // _NS

    c_deg = 512
    c1 = 8192 // hid
    c2 = 8192 // out_dim

    def chunked(c):
        kd = -(-e // (nw * c))
        e_pad = nw * kd * c
        pad_idx = (n + (jnp.arange(e_pad - e, dtype=jnp.int32) % (n_pad - n))
                   ).astype(jnp.int32)
        srcr = jnp.concatenate([edge_index[0], pad_idx]).reshape(nw, kd, c)
        dstr = jnp.concatenate([edge_index[1], pad_idx]).reshape(nw, kd, c)
        return srcr, dstr

    srcr1, dstr1 = chunked(c1)
    srcr2, dstr2 = chunked(c2)
    _, dstrd = chunked(c_deg)

    x_p = jnp.pad(x, ((0, n_pad - n), (0, 0)))
    zeros_hid = jnp.zeros((stripe, hid), jnp.float32)
    zeros_out = jnp.zeros((stripe, out_dim), jnp.float32)
    zeros_deg = jnp.zeros((stripe, _DEGW), jnp.float32)
    ones_deg = jnp.ones((c_deg, _DEGW), jnp.float32)

    degp = _sc_degree(dstrd, zeros_deg, ones_deg, n_pad)
    xw1 = _tc_matmul(x_p, W1)
    y1, dinv = _tc_scale(degp, xw1)
    p1 = _sc_scatter(y1, srcr1, dstr1, zeros_hid, n_pad, hid, c1)
    y2 = _tc_mid(p1, y1, dinv, b1, W2)
    p2 = _sc_scatter(y2, srcr2, dstr2, zeros_out, n_pad, out_dim, c2)
    return _tc_out(p2, y2, dinv, b2, n)
